# baseline (speedup 1.0000x reference)
import functools

import jax
import jax.numpy as jnp
from jax import lax
from jax.experimental import pallas as pl
from jax.experimental.pallas import tpu as pltpu
from jax.experimental.pallas import tpu_sc as plsc

_D = 2048
_E = 16
_K = 2
_BLK = 2048

_NC = 2
_NS = 16
_NW = _NC * _NS
_L = 16


def _logits_body(x_ref, w_ref, logits_ref):
    logits_ref[...] = jax.lax.dot_general(
        x_ref[...], w_ref[...], (((1,), (1,)), ((), ())),
        preferred_element_type=jnp.float32)


def _make_router(bt):
    tok_per_w = bt // _NW
    ngrp = tok_per_w // _L
    mesh = plsc.VectorSubcoreMesh(
        core_axis_name="c", subcore_axis_name="s")

    @functools.partial(
        pl.kernel,
        mesh=mesh,
        compiler_params=pltpu.CompilerParams(needs_layout_passes=False),
        out_type=[
            jax.ShapeDtypeStruct((bt,), jnp.int32),
            jax.ShapeDtypeStruct((bt,), jnp.int32),
            jax.ShapeDtypeStruct((bt,), jnp.float32),
            jax.ShapeDtypeStruct((bt,), jnp.float32),
        ],
        scratch_types=[
            pltpu.VMEM((tok_per_w, _E), jnp.float32),
            pltpu.VMEM((tok_per_w,), jnp.int32),
            pltpu.VMEM((tok_per_w,), jnp.int32),
            pltpu.VMEM((tok_per_w,), jnp.float32),
            pltpu.VMEM((tok_per_w,), jnp.float32),
        ],
    )
    def router(logits_hbm, i1_hbm, i2_hbm, w1_hbm, w2_hbm,
               slab, i1_v, i2_v, w1_v, w2_v):
        wid = lax.axis_index("s") * _NC + lax.axis_index("c")
        base = wid * tok_per_w
        pltpu.sync_copy(logits_hbm.at[pl.ds(base, tok_per_w), :], slab)

        lanes = lax.iota(jnp.int32, _L)
        neg_inf = jnp.full((_L,), -jnp.inf, jnp.float32)
        zeros_i = jnp.zeros((_L,), jnp.int32)

        def group(g, carry):
            rows = g * _L + lanes
            m1 = neg_inf
            m2 = neg_inf
            i1 = zeros_i
            i2 = zeros_i
            for e in range(_E):
                le = plsc.load_gather(
                    slab, [rows, jnp.full((_L,), e, jnp.int32)])
                gt1 = le > m1
                gt2 = le > m2
                m2 = jnp.where(gt1, m1, jnp.where(gt2, le, m2))
                i2 = jnp.where(gt1, i1, jnp.where(gt2, e, i2))
                m1 = jnp.where(gt1, le, m1)
                i1 = jnp.where(gt1, e, i1)
            e2 = jnp.exp(m2 - m1)
            den = 1.0 + e2
            sl = pl.ds(g * _L, _L)
            i1_v[sl] = i1
            i2_v[sl] = i2
            w1_v[sl] = 1.0 / den
            w2_v[sl] = e2 / den
            return carry

        lax.fori_loop(0, ngrp, group, 0)
        pltpu.sync_copy(i1_v, i1_hbm.at[pl.ds(base, tok_per_w)])
        pltpu.sync_copy(i2_v, i2_hbm.at[pl.ds(base, tok_per_w)])
        pltpu.sync_copy(w1_v, w1_hbm.at[pl.ds(base, tok_per_w)])
        pltpu.sync_copy(w2_v, w2_hbm.at[pl.ds(base, tok_per_w)])

    return router


def _tc_logits(x2, W):
    bt, d = x2.shape
    return pl.pallas_call(
        _logits_body,
        grid=(bt // _BLK,),
        in_specs=[
            pl.BlockSpec((_BLK, d), lambda i: (i, 0)),
            pl.BlockSpec((_E, d), lambda i: (0, 0)),
        ],
        out_specs=pl.BlockSpec((_BLK, _E), lambda i: (i, 0)),
        out_shape=jax.ShapeDtypeStruct((bt, _E), jnp.float32),
        compiler_params=pltpu.CompilerParams(
            dimension_semantics=("parallel",)),
    )(x2, W)


@jax.jit
def kernel(x, W):
    b, t, d = x.shape
    bt = b * t
    x2 = x.reshape(bt, d)
    half = bt // 2
    router = _make_router(half)
    logits_a = _tc_logits(x2[:half], W)
    routed_a = router(logits_a)
    logits_b = _tc_logits(x2[half:], W)
    routed_b = router(logits_b)
    i1, i2, w1, w2 = (jnp.concatenate([ra, rb])
                      for ra, rb in zip(routed_a, routed_b))
    logits = jnp.concatenate([logits_a, logits_b])
    idx = jnp.stack([i1, i2], axis=-1)
    wgt = jnp.stack([w1, w2], axis=-1)
    return (idx.reshape(b, t, _K),
            wgt.reshape(b, t, _K),
            logits.reshape(b, t, _E))


# speedup vs baseline: 2.0533x; 2.0533x over previous
import functools

import jax
import jax.numpy as jnp
from jax import lax
from jax.experimental import pallas as pl
from jax.experimental.pallas import tpu as pltpu
from jax.experimental.pallas import tpu_sc as plsc

_D = 2048
_E = 16
_K = 2
_BLK = 2048

_NC = 2
_NS = 16
_NW = _NC * _NS
_L = 16


def _logits_body(x_ref, w_ref, logits_ref):
    logits_ref[...] = jax.lax.dot_general(
        x_ref[...], w_ref[...], (((1,), (1,)), ((), ())),
        preferred_element_type=jnp.float32)


def _make_router(bt):
    tok_per_w = bt // _NW
    ngrp = tok_per_w // _L
    mesh = plsc.VectorSubcoreMesh(
        core_axis_name="c", subcore_axis_name="s")

    @functools.partial(
        pl.kernel,
        mesh=mesh,
        compiler_params=pltpu.CompilerParams(needs_layout_passes=False),
        out_type=[
            jax.ShapeDtypeStruct((bt,), jnp.int32),
            jax.ShapeDtypeStruct((bt,), jnp.int32),
            jax.ShapeDtypeStruct((bt,), jnp.float32),
            jax.ShapeDtypeStruct((bt,), jnp.float32),
        ],
        scratch_types=[
            pltpu.VMEM((tok_per_w, _E), jnp.float32),
            pltpu.VMEM((tok_per_w,), jnp.int32),
            pltpu.VMEM((tok_per_w,), jnp.int32),
            pltpu.VMEM((tok_per_w,), jnp.float32),
            pltpu.VMEM((tok_per_w,), jnp.float32),
        ],
    )
    def router(logits_hbm, i1_hbm, i2_hbm, w1_hbm, w2_hbm,
               slab, i1_v, i2_v, w1_v, w2_v):
        wid = lax.axis_index("s") * _NC + lax.axis_index("c")
        base = wid * tok_per_w
        pltpu.sync_copy(logits_hbm.at[pl.ds(base, tok_per_w), :], slab)

        lanes = lax.iota(jnp.int32, _L)
        neg_inf = jnp.full((_L,), -jnp.inf, jnp.float32)
        zeros_i = jnp.zeros((_L,), jnp.int32)

        def group(g, carry):
            rows = g * _L + lanes
            m1 = neg_inf
            m2 = neg_inf
            i1 = zeros_i
            i2 = zeros_i
            for e in range(_E):
                le = plsc.load_gather(
                    slab, [rows, jnp.full((_L,), e, jnp.int32)])
                gt1 = le > m1
                gt2 = le > m2
                m2 = jnp.where(gt1, m1, jnp.where(gt2, le, m2))
                i2 = jnp.where(gt1, i1, jnp.where(gt2, e, i2))
                m1 = jnp.where(gt1, le, m1)
                i1 = jnp.where(gt1, e, i1)
            e2 = jnp.exp(m2 - m1)
            den = 1.0 + e2
            sl = pl.ds(g * _L, _L)
            i1_v[sl] = i1
            i2_v[sl] = i2
            w1_v[sl] = 1.0 / den
            w2_v[sl] = e2 / den
            return carry

        lax.fori_loop(0, ngrp, group, 0)
        pltpu.sync_copy(i1_v, i1_hbm.at[pl.ds(base, tok_per_w)])
        pltpu.sync_copy(i2_v, i2_hbm.at[pl.ds(base, tok_per_w)])
        pltpu.sync_copy(w1_v, w1_hbm.at[pl.ds(base, tok_per_w)])
        pltpu.sync_copy(w2_v, w2_hbm.at[pl.ds(base, tok_per_w)])

    return router


def _tc_logits(x2, W, ntok, tok0):
    d = x2.shape[1]
    blk0 = tok0 // _BLK
    return pl.pallas_call(
        _logits_body,
        grid=(ntok // _BLK,),
        in_specs=[
            pl.BlockSpec((_BLK, d), lambda i: (i + blk0, 0)),
            pl.BlockSpec((_E, d), lambda i: (0, 0)),
        ],
        out_specs=pl.BlockSpec((_BLK, _E), lambda i: (i, 0)),
        out_shape=jax.ShapeDtypeStruct((ntok, _E), jnp.float32),
        compiler_params=pltpu.CompilerParams(
            dimension_semantics=("parallel",)),
    )(x2, W)


@jax.jit
def kernel(x, W):
    b, t, d = x.shape
    bt = b * t
    x2 = x.reshape(bt, d)
    half = bt // 2
    router = _make_router(half)
    logits_a = _tc_logits(x2, W, half, 0)
    routed_a = router(logits_a)
    logits_b = _tc_logits(x2, W, half, half)
    routed_b = router(logits_b)
    i1, i2, w1, w2 = (jnp.concatenate([ra, rb])
                      for ra, rb in zip(routed_a, routed_b))
    logits = jnp.concatenate([logits_a, logits_b])
    idx = jnp.stack([i1, i2], axis=-1)
    wgt = jnp.stack([w1, w2], axis=-1)
    return (idx.reshape(b, t, _K),
            wgt.reshape(b, t, _K),
            logits.reshape(b, t, _E))


# speedup vs baseline: 2.1194x; 1.0322x over previous
import functools

import jax
import jax.numpy as jnp
from jax import lax
from jax.experimental import pallas as pl
from jax.experimental.pallas import tpu as pltpu
from jax.experimental.pallas import tpu_sc as plsc

_D = 2048
_E = 16
_K = 2
_BLK = 2048

_NC = 2
_NS = 16
_NW = _NC * _NS
_L = 16


def _logits_body(x_ref, w_ref, logits_ref):
    logits_ref[...] = jax.lax.dot_general(
        x_ref[...], w_ref[...], (((1,), (1,)), ((), ())),
        preferred_element_type=jnp.float32)


def _make_router(bt):
    tok_per_w = bt // _NW
    ngrp = tok_per_w // _L
    mesh = plsc.VectorSubcoreMesh(
        core_axis_name="c", subcore_axis_name="s")

    @functools.partial(
        pl.kernel,
        mesh=mesh,
        compiler_params=pltpu.CompilerParams(needs_layout_passes=False),
        out_type=[
            jax.ShapeDtypeStruct((bt,), jnp.int32),
            jax.ShapeDtypeStruct((bt,), jnp.int32),
            jax.ShapeDtypeStruct((bt,), jnp.float32),
            jax.ShapeDtypeStruct((bt,), jnp.float32),
        ],
        scratch_types=[
            pltpu.VMEM((tok_per_w, _E), jnp.float32),
            pltpu.VMEM((tok_per_w,), jnp.int32),
            pltpu.VMEM((tok_per_w,), jnp.int32),
            pltpu.VMEM((tok_per_w,), jnp.float32),
            pltpu.VMEM((tok_per_w,), jnp.float32),
        ],
    )
    def router(logits_hbm, i1_hbm, i2_hbm, w1_hbm, w2_hbm,
               slab, i1_v, i2_v, w1_v, w2_v):
        wid = lax.axis_index("s") * _NC + lax.axis_index("c")
        base = wid * tok_per_w
        pltpu.sync_copy(logits_hbm.at[pl.ds(base, tok_per_w), :], slab)

        lanes = lax.iota(jnp.int32, _L)
        neg_inf = jnp.full((_L,), -jnp.inf, jnp.float32)
        zeros_i = jnp.zeros((_L,), jnp.int32)

        def group(g, carry):
            rows = g * _L + lanes
            m1 = neg_inf
            m2 = neg_inf
            i1 = zeros_i
            i2 = zeros_i
            for e in range(_E):
                le = plsc.load_gather(
                    slab, [rows, jnp.full((_L,), e, jnp.int32)])
                gt1 = le > m1
                gt2 = le > m2
                m2 = jnp.where(gt1, m1, jnp.where(gt2, le, m2))
                i2 = jnp.where(gt1, i1, jnp.where(gt2, e, i2))
                m1 = jnp.where(gt1, le, m1)
                i1 = jnp.where(gt1, e, i1)
            e2 = jnp.exp(m2 - m1)
            den = 1.0 + e2
            sl = pl.ds(g * _L, _L)
            i1_v[sl] = i1
            i2_v[sl] = i2
            w1_v[sl] = 1.0 / den
            w2_v[sl] = e2 / den
            return carry

        lax.fori_loop(0, ngrp, group, 0)
        pltpu.sync_copy(i1_v, i1_hbm.at[pl.ds(base, tok_per_w)])
        pltpu.sync_copy(i2_v, i2_hbm.at[pl.ds(base, tok_per_w)])
        pltpu.sync_copy(w1_v, w1_hbm.at[pl.ds(base, tok_per_w)])
        pltpu.sync_copy(w2_v, w2_hbm.at[pl.ds(base, tok_per_w)])

    return router


def _tc_logits(x2, W, ntok, tok0):
    d = x2.shape[1]
    blk0 = tok0 // _BLK
    return pl.pallas_call(
        _logits_body,
        grid=(ntok // _BLK,),
        in_specs=[
            pl.BlockSpec((_BLK, d), lambda i: (i + blk0, 0)),
            pl.BlockSpec((_E, d), lambda i: (0, 0)),
        ],
        out_specs=pl.BlockSpec((_BLK, _E), lambda i: (i, 0)),
        out_shape=jax.ShapeDtypeStruct((ntok, _E), jnp.float32),
        compiler_params=pltpu.CompilerParams(
            dimension_semantics=("parallel",)),
    )(x2, W)


@jax.jit
def kernel(x, W):
    b, t, d = x.shape
    bt = b * t
    x2 = x.reshape(bt, d)
    logits = _tc_logits(x2, W, bt, 0)
    i1, i2, w1, w2 = _make_router(bt)(logits)
    idx = jnp.stack([i1, i2], axis=-1)
    wgt = jnp.stack([w1, w2], axis=-1)
    return (idx.reshape(b, t, _K),
            wgt.reshape(b, t, _K),
            logits.reshape(b, t, _E))
